# baseline (speedup 1.0000x reference)
import functools

import jax
import jax.numpy as jnp
from jax import lax
from jax.experimental import pallas as pl
from jax.experimental.pallas import tpu as pltpu
from jax.experimental.pallas import tpu_sc as plsc

_R = 64
_K = 32
_L = 16
_NC = 2
_NS = 16
_NW = _NC * _NS


def _mm_body_dual(x_ref, w_ref, full_ref, chunk_ref):
    acts = jnp.dot(x_ref[...], w_ref[...].T,
                   preferred_element_type=jnp.float32)
    full_ref[...] = acts
    chunk_ref[...] = acts


def _mm_body_dual_alias(acts_prev_ref, x_ref, w_ref, full_ref, chunk_ref):
    del acts_prev_ref
    _mm_body_dual(x_ref, w_ref, full_ref, chunk_ref)


def _matmul_chunk(x2, w, acts_prev, row_start, chunk_rows, block_rows):
    n, d = x2.shape
    blocks_per_chunk = chunk_rows // block_rows
    base = row_start // block_rows
    out_shape = (
        jax.ShapeDtypeStruct((n, _R), jnp.float32),
        jax.ShapeDtypeStruct((chunk_rows, _R), jnp.float32),
    )
    out_specs = (
        pl.BlockSpec((block_rows, _R), lambda i: (i + base, 0)),
        pl.BlockSpec((block_rows, _R), lambda i: (i, 0)),
    )
    in_specs = [
        pl.BlockSpec((block_rows, d), lambda i: (i + base, 0)),
        pl.BlockSpec((_R, d), lambda i: (0, 0)),
    ]
    if acts_prev is None:
        return pl.pallas_call(
            _mm_body_dual,
            grid=(blocks_per_chunk,),
            in_specs=in_specs,
            out_specs=out_specs,
            out_shape=out_shape,
        )(x2, w)
    return pl.pallas_call(
        _mm_body_dual_alias,
        grid=(blocks_per_chunk,),
        in_specs=[pl.BlockSpec(memory_space=pl.ANY)] + in_specs,
        out_specs=out_specs,
        out_shape=out_shape,
        input_output_aliases={0: 0},
    )(acts_prev, x2, w)


def _merge16(ka, va, kb, vb):
    rkb = lax.rev(kb, (0,))
    rvb = lax.rev(vb, (0,))
    m = ka >= rkb
    hk = jnp.where(m, ka, rkb)
    hv = jnp.where(m, va, rvb)
    lk = jnp.where(m, rkb, ka)
    lv = jnp.where(m, rvb, va)
    hk, hv = plsc.sort_key_val(hk, hv, descending=True)
    lk, lv = plsc.sort_key_val(lk, lv, descending=True)
    return hk, hv, lk, lv


def _topk_row(k0, k1, k2, k3, iota):
    qs = []
    for q, kq in enumerate((k0, k1, k2, k3)):
        sk, sv = plsc.sort_key_val(kq, iota + (q * _L), descending=True)
        qs.append((sk, sv))
    ahk, ahv, alk, alv = _merge16(qs[0][0], qs[0][1], qs[1][0], qs[1][1])
    bhk, bhv, blk, blv = _merge16(qs[2][0], qs[2][1], qs[3][0], qs[3][1])
    rb0 = lax.rev(blk, (0,))
    rv0 = lax.rev(blv, (0,))
    rb1 = lax.rev(bhk, (0,))
    rv1 = lax.rev(bhv, (0,))
    m0 = ahk >= rb0
    h0k = jnp.where(m0, ahk, rb0)
    h0v = jnp.where(m0, ahv, rv0)
    m1 = alk >= rb1
    h1k = jnp.where(m1, alk, rb1)
    h1v = jnp.where(m1, alv, rv1)
    mm = h0k >= h1k
    hhk = jnp.where(mm, h0k, h1k)
    hhv = jnp.where(mm, h0v, h1v)
    hlk = jnp.where(mm, h1k, h0k)
    hlv = jnp.where(mm, h1v, h0v)
    hhk, hhv = plsc.sort_key_val(hhk, hhv, descending=True)
    hlk, hlv = plsc.sort_key_val(hlk, hlv, descending=True)
    return hhk, hhv, hlk, hlv


def _topk_body(acts_hbm, idx_hbm, val_hbm, acts_v, idx_v, val_v,
               rows_w, out_col0):
    wid = lax.axis_index("s") * _NC + lax.axis_index("c")
    pltpu.sync_copy(acts_hbm.at[pl.ds(wid * rows_w, rows_w)], acts_v)
    iota = lax.broadcasted_iota(jnp.int32, (_L,), 0)

    def body2(r):
        k0 = acts_v[r, pl.ds(0, _L)]
        k1 = acts_v[r, pl.ds(_L, _L)]
        k2 = acts_v[r, pl.ds(2 * _L, _L)]
        k3 = acts_v[r, pl.ds(3 * _L, _L)]
        hhk, hhv, hlk, hlv = _topk_row(k0, k1, k2, k3, iota)
        rcol = jnp.zeros((_L,), jnp.int32) + r
        plsc.store_scatter(val_v, [iota, rcol], hhk)
        plsc.store_scatter(val_v, [iota + _L, rcol], hlk)
        plsc.store_scatter(idx_v, [iota, rcol], hhv)
        plsc.store_scatter(idx_v, [iota + _L, rcol], hlv)

    plsc.parallel_loop(0, rows_w, 1, unroll=8, carry=None)(body2)
    col = out_col0 + wid * rows_w
    pltpu.sync_copy(idx_v, idx_hbm.at[:, pl.ds(col, rows_w)])
    pltpu.sync_copy(val_v, val_hbm.at[:, pl.ds(col, rows_w)])
    return wid


def _sc_topk_chunk(acts, idx_ref, val_ref, col0):
    n_rows = acts.shape[0]
    rows_w = n_rows // _NW
    mesh = plsc.VectorSubcoreMesh(core_axis_name="c", subcore_axis_name="s",
                                  num_cores=_NC, num_subcores=_NS)

    @functools.partial(
        pl.kernel,
        out_type=(),
        mesh=mesh,
        scratch_types=[
            pltpu.VMEM((rows_w, _R), jnp.float32),
            pltpu.VMEM((_K, rows_w), jnp.int32),
            pltpu.VMEM((_K, rows_w), jnp.float32),
        ],
        compiler_params=pltpu.CompilerParams(needs_layout_passes=False),
    )
    def k(acts_hbm, idx_hbm, val_hbm, acts_v, idx_v, val_v):
        _topk_body(acts_hbm, idx_hbm, val_hbm, acts_v, idx_v, val_v,
                   rows_w, col0)

    return k(acts, idx_ref, val_ref)


_CHUNK_ROWS = (4096, 4096, 4096, 4096)
_BLOCK_ROWS = 1024


@jax.jit
def kernel(x, W_A):
    b, s, d = x.shape
    n = b * s
    x2 = x.reshape(n, d)
    idx_ref = jax.new_ref(jnp.zeros((_K, n), jnp.int32))
    val_ref = jax.new_ref(jnp.zeros((_K, n), jnp.float32))
    acts = None
    row = 0
    for chunk in _CHUNK_ROWS:
        acts, acts_c = _matmul_chunk(x2, W_A, acts, row, chunk, _BLOCK_ROWS)
        _sc_topk_chunk(acts_c, idx_ref, val_ref, row)
        row += chunk
    idx_t = jax.freeze(idx_ref)
    val_t = jax.freeze(val_ref)
    return idx_t.T, val_t.T, acts.reshape(b, s, _R)


# speedup vs baseline: 1.0088x; 1.0088x over previous
import functools

import jax
import jax.numpy as jnp
from jax import lax
from jax.experimental import pallas as pl
from jax.experimental.pallas import tpu as pltpu
from jax.experimental.pallas import tpu_sc as plsc

_R = 64
_K = 32
_L = 16
_NC = 2
_NS = 16
_NW = _NC * _NS


def _mm_body_dual(x_ref, w_ref, full_ref, chunk_ref):
    acts = jnp.dot(x_ref[...], w_ref[...].T,
                   preferred_element_type=jnp.float32)
    full_ref[...] = acts
    chunk_ref[...] = acts


def _mm_body_dual_alias(acts_prev_ref, x_ref, w_ref, full_ref, chunk_ref):
    del acts_prev_ref
    _mm_body_dual(x_ref, w_ref, full_ref, chunk_ref)


def _matmul_chunk(x2, w, acts_prev, row_start, chunk_rows, block_rows):
    n, d = x2.shape
    blocks_per_chunk = chunk_rows // block_rows
    base = row_start // block_rows
    out_shape = (
        jax.ShapeDtypeStruct((n, _R), jnp.float32),
        jax.ShapeDtypeStruct((chunk_rows, _R), jnp.float32),
    )
    out_specs = (
        pl.BlockSpec((block_rows, _R), lambda i: (i + base, 0)),
        pl.BlockSpec((block_rows, _R), lambda i: (i, 0)),
    )
    in_specs = [
        pl.BlockSpec((block_rows, d), lambda i: (i + base, 0)),
        pl.BlockSpec((_R, d), lambda i: (0, 0)),
    ]
    if acts_prev is None:
        return pl.pallas_call(
            _mm_body_dual,
            grid=(blocks_per_chunk,),
            in_specs=in_specs,
            out_specs=out_specs,
            out_shape=out_shape,
        )(x2, w)
    return pl.pallas_call(
        _mm_body_dual_alias,
        grid=(blocks_per_chunk,),
        in_specs=[pl.BlockSpec(memory_space=pl.ANY)] + in_specs,
        out_specs=out_specs,
        out_shape=out_shape,
        input_output_aliases={0: 0},
    )(acts_prev, x2, w)


def _merge16(ka, va, kb, vb):
    rkb = lax.rev(kb, (0,))
    rvb = lax.rev(vb, (0,))
    m = ka >= rkb
    hk = jnp.where(m, ka, rkb)
    hv = jnp.where(m, va, rvb)
    lk = jnp.where(m, rkb, ka)
    lv = jnp.where(m, rvb, va)
    hk, hv = plsc.sort_key_val(hk, hv, descending=True)
    lk, lv = plsc.sort_key_val(lk, lv, descending=True)
    return hk, hv, lk, lv


def _topk_row(k0, k1, k2, k3, iota):
    qs = []
    for q, kq in enumerate((k0, k1, k2, k3)):
        sk, sv = plsc.sort_key_val(kq, iota + (q * _L), descending=True)
        qs.append((sk, sv))
    ahk, ahv, alk, alv = _merge16(qs[0][0], qs[0][1], qs[1][0], qs[1][1])
    bhk, bhv, blk, blv = _merge16(qs[2][0], qs[2][1], qs[3][0], qs[3][1])
    rb0 = lax.rev(blk, (0,))
    rv0 = lax.rev(blv, (0,))
    rb1 = lax.rev(bhk, (0,))
    rv1 = lax.rev(bhv, (0,))
    m0 = ahk >= rb0
    h0k = jnp.where(m0, ahk, rb0)
    h0v = jnp.where(m0, ahv, rv0)
    m1 = alk >= rb1
    h1k = jnp.where(m1, alk, rb1)
    h1v = jnp.where(m1, alv, rv1)
    mm = h0k >= h1k
    hhk = jnp.where(mm, h0k, h1k)
    hhv = jnp.where(mm, h0v, h1v)
    hlk = jnp.where(mm, h1k, h0k)
    hlv = jnp.where(mm, h1v, h0v)
    hhk, hhv = plsc.sort_key_val(hhk, hhv, descending=True)
    hlk, hlv = plsc.sort_key_val(hlk, hlv, descending=True)
    return hhk, hhv, hlk, hlv


def _topk_body(acts_hbm, idx_hbm, val_hbm, acts_v, idx_v, val_v,
               rows_w, out_col0):
    wid = lax.axis_index("s") * _NC + lax.axis_index("c")
    pltpu.sync_copy(acts_hbm.at[pl.ds(wid * rows_w, rows_w)], acts_v)
    iota = lax.broadcasted_iota(jnp.int32, (_L,), 0)

    def body2(r):
        k0 = acts_v[r, pl.ds(0, _L)]
        k1 = acts_v[r, pl.ds(_L, _L)]
        k2 = acts_v[r, pl.ds(2 * _L, _L)]
        k3 = acts_v[r, pl.ds(3 * _L, _L)]
        hhk, hhv, hlk, hlv = _topk_row(k0, k1, k2, k3, iota)
        rcol = jnp.zeros((_L,), jnp.int32) + r
        plsc.store_scatter(val_v, [iota, rcol], hhk)
        plsc.store_scatter(val_v, [iota + _L, rcol], hlk)
        plsc.store_scatter(idx_v, [iota, rcol], hhv)
        plsc.store_scatter(idx_v, [iota + _L, rcol], hlv)

    plsc.parallel_loop(0, rows_w, 1, unroll=4, carry=None)(body2)
    col = out_col0 + wid * rows_w
    pltpu.sync_copy(idx_v, idx_hbm.at[:, pl.ds(col, rows_w)])
    pltpu.sync_copy(val_v, val_hbm.at[:, pl.ds(col, rows_w)])
    return wid


def _sc_topk_chunk(acts, idx_ref, val_ref, col0):
    n_rows = acts.shape[0]
    rows_w = n_rows // _NW
    mesh = plsc.VectorSubcoreMesh(core_axis_name="c", subcore_axis_name="s",
                                  num_cores=_NC, num_subcores=_NS)

    @functools.partial(
        pl.kernel,
        out_type=(),
        mesh=mesh,
        scratch_types=[
            pltpu.VMEM((rows_w, _R), jnp.float32),
            pltpu.VMEM((_K, rows_w), jnp.int32),
            pltpu.VMEM((_K, rows_w), jnp.float32),
        ],
        compiler_params=pltpu.CompilerParams(needs_layout_passes=False),
    )
    def k(acts_hbm, idx_hbm, val_hbm, acts_v, idx_v, val_v):
        _topk_body(acts_hbm, idx_hbm, val_hbm, acts_v, idx_v, val_v,
                   rows_w, col0)

    return k(acts, idx_ref, val_ref)


_CHUNK_ROWS = (4096, 4096, 4096, 4096)
_BLOCK_ROWS = 1024


@jax.jit
def kernel(x, W_A):
    b, s, d = x.shape
    n = b * s
    x2 = x.reshape(n, d)
    idx_ref = jax.new_ref(jnp.zeros((_K, n), jnp.int32))
    val_ref = jax.new_ref(jnp.zeros((_K, n), jnp.float32))
    acts = None
    row = 0
    for chunk in _CHUNK_ROWS:
        acts, acts_c = _matmul_chunk(x2, W_A, acts, row, chunk, _BLOCK_ROWS)
        _sc_topk_chunk(acts_c, idx_ref, val_ref, row)
        row += chunk
    idx_t = jax.freeze(idx_ref)
    val_t = jax.freeze(val_ref)
    return idx_t.T, val_t.T, acts.reshape(b, s, _R)
